# Initial kernel scaffold; baseline (speedup 1.0000x reference)
import jax
import jax.numpy as jnp
from jax.experimental import pallas as pl


def kernel(x, Wg, bg, W1, b1, W2, b2):
    raise NotImplementedError("write your pallas kernel here")


# speedup vs baseline: 5.9344x; 5.9344x over previous
import functools

import jax
import jax.numpy as jnp
from jax.experimental import pallas as pl
from jax.experimental.pallas import tpu as pltpu

D = 768
E = 16
H = 4 * D
T = 4096
BT = 128
NB = T // BT + E
TP = NB * BT
GB = 512
EPAD = 128


def _gate_body(x_ref, wg_ref, bg_ref, eid_ref):
    logits = jnp.dot(x_ref[...], wg_ref[...],
                     preferred_element_type=jnp.float32) + bg_ref[...]
    m = jnp.max(logits, axis=1, keepdims=True)
    idx = jax.lax.broadcasted_iota(jnp.int32, (GB, EPAD), 1)
    eid = jnp.min(jnp.where(logits >= m, idx, EPAD), axis=1)
    eid_ref[...] = eid.reshape(1, 1, GB)


def _gate(xf, Wg, bg):
    wg_pad = jnp.zeros((D, EPAD), jnp.float32).at[:, :E].set(Wg)
    bg_pad = jnp.full((1, EPAD), -1e30, jnp.float32).at[0, :E].set(bg)
    eid = pl.pallas_call(
        _gate_body,
        grid=(T // GB,),
        in_specs=[
            pl.BlockSpec((GB, D), lambda b: (b, 0)),
            pl.BlockSpec((D, EPAD), lambda b: (0, 0)),
            pl.BlockSpec((1, EPAD), lambda b: (0, 0)),
        ],
        out_specs=pl.BlockSpec((1, 1, GB), lambda b: (b, 0, 0)),
        out_shape=jax.ShapeDtypeStruct((T // GB, 1, GB), jnp.int32),
    )(xf, wg_pad, bg_pad)
    return eid.reshape(T)


def _ffn_body(be_ref, nv_ref, x_ref, w1_ref, b1_ref, w2_ref, b2_ref, out_ref):
    b = pl.program_id(0)

    @pl.when(b < nv_ref[0])
    def _():
        xb = x_ref[...]
        out_ref[...] = jnp.broadcast_to(b2_ref[0, 0], (BT, D))
        nh = 4
        bh = H // nh
        for j in range(nh):
            ht = jnp.dot(xb, w1_ref[0, :, j * bh:(j + 1) * bh],
                         preferred_element_type=jnp.float32) + b1_ref[0, 0, j * bh:(j + 1) * bh]
            ht = ht * 0.5 * (1.0 + jax.lax.erf(ht * 0.7071067811865476))
            out_ref[...] += jnp.dot(ht, w2_ref[0, j * bh:(j + 1) * bh, :],
                                    preferred_element_type=jnp.float32)


def _ffn(x_sorted, block_expert, nvalid, W1, b1, W2, b2):
    grid_spec = pltpu.PrefetchScalarGridSpec(
        num_scalar_prefetch=2,
        grid=(NB,),
        in_specs=[
            pl.BlockSpec((BT, D), lambda b, be, nv: (b, 0)),
            pl.BlockSpec((1, D, H), lambda b, be, nv: (be[b], 0, 0)),
            pl.BlockSpec((1, 1, H), lambda b, be, nv: (be[b], 0, 0)),
            pl.BlockSpec((1, H, D), lambda b, be, nv: (be[b], 0, 0)),
            pl.BlockSpec((1, 1, D), lambda b, be, nv: (be[b], 0, 0)),
        ],
        out_specs=pl.BlockSpec((BT, D), lambda b, be, nv: (b, 0)),
    )
    return pl.pallas_call(
        _ffn_body,
        grid_spec=grid_spec,
        out_shape=jax.ShapeDtypeStruct((TP, D), jnp.float32),
    )(block_expert, nvalid, x_sorted, W1, b1.reshape(E, 1, H), W2,
      b2.reshape(E, 1, D))


def kernel(x, Wg, bg, W1, b1, W2, b2):
    B, S, _ = x.shape
    xf = x.reshape(T, D)
    eid = _gate(xf, Wg, bg)

    counts = jnp.zeros((E,), jnp.int32).at[eid].add(1)
    rc = ((counts + BT - 1) // BT) * BT
    oend = jnp.cumsum(rc)
    ostart = oend - rc
    nvalid = (oend[-1] // BT).reshape(1)

    sort_idx = jnp.argsort(eid)
    se = eid[sort_idx]
    cstart = jnp.cumsum(counts) - counts
    j = jnp.arange(T, dtype=jnp.int32)
    slot_sorted = (ostart[se] + (j - cstart[se])).astype(jnp.int32)
    slot = jnp.zeros((T,), jnp.int32).at[sort_idx].set(slot_sorted)

    bstart = jnp.arange(NB, dtype=jnp.int32) * BT
    bs2 = jnp.minimum(bstart, oend[-1] - 1)
    block_expert = jnp.searchsorted(oend, bs2, side="right").astype(jnp.int32)

    gidx = jnp.zeros((TP,), jnp.int32).at[slot].set(j)
    x_sorted = xf[gidx]

    out_sorted = _ffn(x_sorted, block_expert, nvalid, W1, b1, W2, b2)
    out = out_sorted[slot]
    return out.reshape(B, S, D)
